# baseline (speedup 1.0000x reference)
import jax
import jax.numpy as jnp
from jax.experimental import pallas as pl


def _add_kernel(x_ref, t_ref, o_ref):
    o_ref[...] = x_ref[...] + t_ref[...]


def kernel(inputs, pos_table):
    batch, positions, dim = inputs.shape
    return pl.pallas_call(
        _add_kernel,
        grid=(batch,),
        in_specs=[
            pl.BlockSpec((1, positions, dim), lambda b: (b, 0, 0)),
            pl.BlockSpec((positions, dim), lambda b: (0, 0)),
        ],
        out_specs=pl.BlockSpec((1, positions, dim), lambda b: (b, 0, 0)),
        out_shape=jax.ShapeDtypeStruct(inputs.shape, inputs.dtype),
    )(inputs, pos_table)


# speedup vs baseline: 1.0106x; 1.0106x over previous
import jax
import jax.numpy as jnp
from jax.experimental import pallas as pl


def _add_kernel(x_ref, t_ref, o_ref):
    o_ref[...] = x_ref[...] + t_ref[...]


def kernel(inputs, pos_table):
    batch, positions, dim = inputs.shape
    bb = 4
    return pl.pallas_call(
        _add_kernel,
        grid=(batch // bb,),
        in_specs=[
            pl.BlockSpec((bb, positions, dim), lambda b: (b, 0, 0)),
            pl.BlockSpec((positions, dim), lambda b: (0, 0)),
        ],
        out_specs=pl.BlockSpec((bb, positions, dim), lambda b: (b, 0, 0)),
        out_shape=jax.ShapeDtypeStruct(inputs.shape, inputs.dtype),
    )(inputs, pos_table)
